# baseline (speedup 1.0000x reference)
import functools

import jax
import jax.numpy as jnp
from jax import lax
from jax.experimental import pallas as pl
from jax.experimental.pallas import tpu as pltpu
from jax.experimental.pallas import tpu_sc as plsc

N = 10000
D = 128
G = 64
NC, NS = 2, 16
NW = NC * NS
CHUNK = 128
NPAD = 10240
ROWS_PER_SUB = NPAD // NS


def _fill(ref, vec16):
    (n,) = ref.shape

    def body(i, _):
        ref[pl.ds(i * 16, 16)] = vec16
        return 0

    lax.fori_loop(0, n // 16, body, 0)


def _zero_rows(ref):
    r, c = ref.shape
    z = jnp.zeros((16,), jnp.float32)

    def body(i, _):
        for j in range(c // 16):
            ref[i, pl.ds(j * 16, 16)] = z
        return 0

    lax.fori_loop(0, r, body, 0)


@functools.lru_cache(maxsize=None)
def _make_deg(epad):
    e_per_w = epad // NW
    n_chunks = e_per_w // CHUNK
    mesh = plsc.VectorSubcoreMesh(core_axis_name="c", subcore_axis_name="s")

    @functools.partial(
        pl.kernel,
        mesh=mesh,
        out_type=jax.ShapeDtypeStruct((NC, NPAD), jnp.float32),
        scratch_types=[
            pltpu.VMEM((CHUNK,), jnp.int32),
            pltpu.VMEM((CHUNK,), jnp.float32),
            pltpu.VMEM((ROWS_PER_SUB,), jnp.float32),
            pltpu.VMEM_SHARED((NPAD,), jnp.float32),
        ],
    )
    def deg_kernel(dst_hbm, out_hbm, idx_v, ones_v, buf_v, acc_sh):
        c = lax.axis_index("c")
        s = lax.axis_index("s")
        wid = s * NC + c
        _fill(ones_v, jnp.full((16,), 1.0, jnp.float32))
        _fill(buf_v, jnp.zeros((16,), jnp.float32))
        pltpu.sync_copy(buf_v, acc_sh.at[pl.ds(s * ROWS_PER_SUB, ROWS_PER_SUB)])
        plsc.subcore_barrier()

        base = wid * e_per_w

        def body(i, _):
            off = base + i * CHUNK
            pltpu.sync_copy(dst_hbm.at[pl.ds(off, CHUNK)], idx_v)
            pltpu.sync_copy(ones_v, acc_sh.at[idx_v], add=True)
            return 0

        lax.fori_loop(0, n_chunks, body, 0)
        plsc.subcore_barrier()
        sl = pl.ds(s * ROWS_PER_SUB, ROWS_PER_SUB)
        pltpu.sync_copy(acc_sh.at[sl], buf_v)
        pltpu.sync_copy(buf_v, out_hbm.at[c, sl])

    return deg_kernel


AGG_CHUNK = 128
PHASES = 4


@functools.lru_cache(maxsize=None)
def _make_agg(epad):
    n_chunks = epad // AGG_CHUNK // NW
    cpp = n_chunks // PHASES
    mesh = plsc.VectorSubcoreMesh(core_axis_name="c", subcore_axis_name="s")

    @functools.partial(
        pl.kernel,
        mesh=mesh,
        out_type=jax.ShapeDtypeStruct((NC, NPAD, D), jnp.float32),
        scratch_types=[
            pltpu.VMEM((cpp, 2, AGG_CHUNK), jnp.int32),
            pltpu.VMEM((2, AGG_CHUNK, D), jnp.float32),
            pltpu.VMEM_SHARED((NPAD, D), jnp.float32),
            pltpu.SemaphoreType.DMA,
            pltpu.SemaphoreType.DMA,
            pltpu.SemaphoreType.DMA,
            pltpu.SemaphoreType.DMA,
        ],
    )
    def agg_kernel(g_hbm, eidx_hbm, out_hbm, eidx_v, bufs_v, acc_sh,
                   gsem0, gsem1, ssem0, ssem1):
        c = lax.axis_index("c")
        s = lax.axis_index("s")
        wid = s * NC + c
        cbase = wid * n_chunks

        _zero_rows(bufs_v.at[0])
        for k in range(ROWS_PER_SUB // AGG_CHUNK):
            pltpu.sync_copy(
                bufs_v.at[0], acc_sh.at[pl.ds(s * ROWS_PER_SUB + k * AGG_CHUNK, AGG_CHUNK)])
        plsc.subcore_barrier()

        gsems = (gsem0, gsem1)
        ssems = (ssem0, ssem1)

        def gather(chunk, b):
            pltpu.async_copy(g_hbm.at[eidx_v.at[chunk, 0]], bufs_v.at[b], gsems[b])

        def gwait(chunk, b):
            pltpu.make_async_copy(g_hbm.at[eidx_v.at[chunk, 0]], bufs_v.at[b],
                                  gsems[b]).wait()

        def scatter(chunk, b):
            pltpu.async_copy(bufs_v.at[b], acc_sh.at[eidx_v.at[chunk, 1]],
                             ssems[b], add=True)

        def swait(chunk, b):
            pltpu.make_async_copy(bufs_v.at[b], acc_sh.at[eidx_v.at[chunk, 1]],
                                  ssems[b]).wait()

        for ph in range(PHASES):
            pltpu.sync_copy(eidx_hbm.at[pl.ds(cbase + ph * cpp, cpp)], eidx_v)
            gather(0, 0)

            def body(j, _):
                for b in range(2):
                    chunk = 2 * j + b
                    gwait(chunk, b)
                    scatter(chunk, b)

                    @pl.when(chunk >= 1)
                    def _():
                        swait(chunk - 1, (b + 1) % 2)

                    @pl.when(chunk + 1 < cpp)
                    def _():
                        gather(chunk + 1, (b + 1) % 2)

                return 0

            lax.fori_loop(0, cpp // 2, body, 0)
            swait(cpp - 1, (cpp - 1) % 2)
        plsc.subcore_barrier()
        for k in range(ROWS_PER_SUB // AGG_CHUNK):
            sl = pl.ds(s * ROWS_PER_SUB + k * AGG_CHUNK, AGG_CHUNK)
            pltpu.sync_copy(acc_sh.at[sl], bufs_v.at[k % 2])
            pltpu.sync_copy(bufs_v.at[k % 2], out_hbm.at[c, sl])

    return agg_kernel


def _dinv_g1_body(degp_ref, x_ref, w1_ref, dinv_ref, g1_ref):
    deg = degp_ref[0] + degp_ref[1] + 1.0
    dinv = lax.rsqrt(deg)
    dinv_ref[...] = dinv
    h = lax.dot_general(x_ref[...], w1_ref[...], (((1,), (1,)), ((), ())),
                        preferred_element_type=jnp.float32)
    g1_ref[...] = h * dinv[:N]


def _mid_body(p_ref, g_ref, dinv_ref, b_ref, w_ref, gout_ref):
    dinv = dinv_ref[...][:N]
    ssum = (p_ref[0, :N, :] + p_ref[1, :N, :] + g_ref[...]) * dinv
    h = jnp.maximum(ssum + b_ref[...], 0.0)
    g2 = lax.dot_general(h, w_ref[...], (((1,), (1,)), ((), ())),
                         preferred_element_type=jnp.float32)
    gout_ref[...] = g2 * dinv


def _final_body(p_ref, g_ref, dinv_ref, b_ref, batch_ref, wfc_ref, bfc_ref, out_ref):
    dinv = dinv_ref[...][:N]
    h = jnp.maximum((p_ref[0, :N, :] + p_ref[1, :N, :] + g_ref[...]) * dinv + b_ref[...], 0.0)
    seg = lax.broadcasted_iota(jnp.int32, (G, N), 0)
    oh = (seg == batch_ref[...]).astype(jnp.float32)
    sums = lax.dot_general(oh, h, (((1,), (0,)), ((), ())),
                           preferred_element_type=jnp.float32)
    cnt = jnp.sum(oh, axis=1, keepdims=True)
    pooled = sums / jnp.maximum(cnt, 1.0)
    o = lax.dot_general(pooled, wfc_ref[...], (((1,), (1,)), ((), ())),
                        preferred_element_type=jnp.float32)
    out_ref[...] = jnp.maximum(o + bfc_ref[...], 0.0)


def kernel(x, edge_index, batch, W1, b1, W2, b2, Wfc, bfc):
    e = edge_index.shape[1]
    step = NW * AGG_CHUNK * 2 * PHASES
    epad = ((e + step - 1) // step) * step
    pad = epad - e
    src = jnp.concatenate([edge_index[0], jnp.zeros((pad,), jnp.int32)])
    dst = jnp.concatenate([edge_index[1], jnp.full((pad,), N, jnp.int32)])
    eidx = jnp.concatenate(
        [src.reshape(epad // AGG_CHUNK, 1, AGG_CHUNK),
         dst.reshape(epad // AGG_CHUNK, 1, AGG_CHUNK)], axis=1)

    degp = _make_deg(epad)(dst)
    degp3 = degp.reshape(NC, NPAD, 1)

    dinv, g1 = pl.pallas_call(
        _dinv_g1_body,
        out_shape=[
            jax.ShapeDtypeStruct((NPAD, 1), jnp.float32),
            jax.ShapeDtypeStruct((N, D), jnp.float32),
        ],
    )(degp3, x, W1)

    agg = _make_agg(epad)
    p1 = agg(g1, eidx)

    g2 = pl.pallas_call(
        _mid_body,
        out_shape=jax.ShapeDtypeStruct((N, D), jnp.float32),
    )(p1, g1, dinv, b1.reshape(1, D), W2)

    p2 = agg(g2, eidx)

    out = pl.pallas_call(
        _final_body,
        out_shape=jax.ShapeDtypeStruct((G, D), jnp.float32),
    )(p2, g2, dinv, b2.reshape(1, D), batch.reshape(1, N), Wfc, bfc.reshape(1, D))
    return out


# speedup vs baseline: 1.2978x; 1.2978x over previous
import functools

import jax
import jax.numpy as jnp
from jax import lax
from jax.experimental import pallas as pl
from jax.experimental.pallas import tpu as pltpu
from jax.experimental.pallas import tpu_sc as plsc

N = 10000
D = 128
G = 64
NC, NS = 2, 16
NW = NC * NS
CHUNK = 128
NPAD = 10240
ROWS_PER_SUB = NPAD // NS


def _fill(ref, vec16):
    (n,) = ref.shape

    def body(i, _):
        ref[pl.ds(i * 16, 16)] = vec16
        return 0

    lax.fori_loop(0, n // 16, body, 0)


def _zero_rows(ref):
    r, c = ref.shape
    z = jnp.zeros((16,), jnp.float32)

    def body(i, _):
        for j in range(c // 16):
            ref[i, pl.ds(j * 16, 16)] = z
        return 0

    lax.fori_loop(0, r, body, 0)


@functools.lru_cache(maxsize=None)
def _make_deg(epad):
    e_per_w = epad // NW
    n_chunks = e_per_w // CHUNK
    mesh = plsc.VectorSubcoreMesh(core_axis_name="c", subcore_axis_name="s")

    @functools.partial(
        pl.kernel,
        mesh=mesh,
        out_type=jax.ShapeDtypeStruct((NC, NPAD), jnp.float32),
        scratch_types=[
            pltpu.VMEM((CHUNK,), jnp.int32),
            pltpu.VMEM((CHUNK,), jnp.float32),
            pltpu.VMEM((ROWS_PER_SUB,), jnp.float32),
            pltpu.VMEM_SHARED((NPAD,), jnp.float32),
        ],
    )
    def deg_kernel(dst_hbm, out_hbm, idx_v, ones_v, buf_v, acc_sh):
        c = lax.axis_index("c")
        s = lax.axis_index("s")
        wid = s * NC + c
        _fill(ones_v, jnp.full((16,), 1.0, jnp.float32))
        _fill(buf_v, jnp.zeros((16,), jnp.float32))
        pltpu.sync_copy(buf_v, acc_sh.at[pl.ds(s * ROWS_PER_SUB, ROWS_PER_SUB)])
        plsc.subcore_barrier()

        base = wid * e_per_w

        def body(i, _):
            off = base + i * CHUNK
            pltpu.sync_copy(dst_hbm.at[pl.ds(off, CHUNK)], idx_v)
            pltpu.sync_copy(ones_v, acc_sh.at[idx_v], add=True)
            return 0

        lax.fori_loop(0, n_chunks, body, 0)
        plsc.subcore_barrier()
        sl = pl.ds(s * ROWS_PER_SUB, ROWS_PER_SUB)
        pltpu.sync_copy(acc_sh.at[sl], buf_v)
        pltpu.sync_copy(buf_v, out_hbm.at[c, sl])

    return deg_kernel


@functools.lru_cache(maxsize=None)
def _make_agg(epad):
    e_per_w = epad // NW
    n_chunks = e_per_w // CHUNK
    mesh = plsc.VectorSubcoreMesh(core_axis_name="c", subcore_axis_name="s")

    @functools.partial(
        pl.kernel,
        mesh=mesh,
        out_type=jax.ShapeDtypeStruct((NC, NPAD, D), jnp.float32),
        scratch_types=[
            pltpu.VMEM((CHUNK,), jnp.int32),
            pltpu.VMEM((CHUNK,), jnp.int32),
            pltpu.VMEM((CHUNK, D), jnp.float32),
            pltpu.VMEM((CHUNK, D), jnp.float32),
            pltpu.VMEM_SHARED((NPAD, D), jnp.float32),
        ],
    )
    def agg_kernel(g_hbm, src_hbm, dst_hbm, out_hbm, sidx_v, didx_v, rows_v, zbuf_v, acc_sh):
        c = lax.axis_index("c")
        s = lax.axis_index("s")
        wid = s * NC + c
        _zero_rows(zbuf_v)
        for k in range(ROWS_PER_SUB // CHUNK):
            pltpu.sync_copy(zbuf_v, acc_sh.at[pl.ds(s * ROWS_PER_SUB + k * CHUNK, CHUNK)])
        plsc.subcore_barrier()

        base = wid * e_per_w

        def body(i, _):
            off = base + i * CHUNK
            pltpu.sync_copy(src_hbm.at[pl.ds(off, CHUNK)], sidx_v)
            pltpu.sync_copy(dst_hbm.at[pl.ds(off, CHUNK)], didx_v)
            pltpu.sync_copy(g_hbm.at[sidx_v], rows_v)
            pltpu.sync_copy(rows_v, acc_sh.at[didx_v], add=True)
            return 0

        lax.fori_loop(0, n_chunks, body, 0)
        plsc.subcore_barrier()
        for k in range(ROWS_PER_SUB // CHUNK):
            sl = pl.ds(s * ROWS_PER_SUB + k * CHUNK, CHUNK)
            pltpu.sync_copy(acc_sh.at[sl], rows_v)
            pltpu.sync_copy(rows_v, out_hbm.at[c, sl])

    return agg_kernel


def _dinv_g1_body(degp_ref, x_ref, w1_ref, dinv_ref, g1_ref):
    deg = degp_ref[0] + degp_ref[1] + 1.0
    dinv = lax.rsqrt(deg)
    dinv_ref[...] = dinv
    h = lax.dot_general(x_ref[...], w1_ref[...], (((1,), (1,)), ((), ())),
                        preferred_element_type=jnp.float32)
    g1_ref[...] = h * dinv[:N]


def _mid_body(p_ref, g_ref, dinv_ref, b_ref, w_ref, gout_ref):
    dinv = dinv_ref[...][:N]
    ssum = (p_ref[0, :N, :] + p_ref[1, :N, :] + g_ref[...]) * dinv
    h = jnp.maximum(ssum + b_ref[...], 0.0)
    g2 = lax.dot_general(h, w_ref[...], (((1,), (1,)), ((), ())),
                         preferred_element_type=jnp.float32)
    gout_ref[...] = g2 * dinv


def _final_body(p_ref, g_ref, dinv_ref, b_ref, batch_ref, wfc_ref, bfc_ref, out_ref):
    dinv = dinv_ref[...][:N]
    h = jnp.maximum((p_ref[0, :N, :] + p_ref[1, :N, :] + g_ref[...]) * dinv + b_ref[...], 0.0)
    seg = lax.broadcasted_iota(jnp.int32, (G, N), 0)
    oh = (seg == batch_ref[...]).astype(jnp.float32)
    sums = lax.dot_general(oh, h, (((1,), (0,)), ((), ())),
                           preferred_element_type=jnp.float32)
    cnt = jnp.sum(oh, axis=1, keepdims=True)
    pooled = sums / jnp.maximum(cnt, 1.0)
    o = lax.dot_general(pooled, wfc_ref[...], (((1,), (1,)), ((), ())),
                        preferred_element_type=jnp.float32)
    out_ref[...] = jnp.maximum(o + bfc_ref[...], 0.0)


def kernel(x, edge_index, batch, W1, b1, W2, b2, Wfc, bfc):
    e = edge_index.shape[1]
    step = NW * CHUNK
    epad = ((e + step - 1) // step) * step
    pad = epad - e
    src = jnp.concatenate([edge_index[0], jnp.zeros((pad,), jnp.int32)])
    dst = jnp.concatenate([edge_index[1], jnp.full((pad,), N, jnp.int32)])

    degp = _make_deg(epad)(dst)
    degp3 = degp.reshape(NC, NPAD, 1)

    dinv, g1 = pl.pallas_call(
        _dinv_g1_body,
        out_shape=[
            jax.ShapeDtypeStruct((NPAD, 1), jnp.float32),
            jax.ShapeDtypeStruct((N, D), jnp.float32),
        ],
    )(degp3, x, W1)

    agg = _make_agg(epad)
    p1 = agg(g1, src, dst)

    g2 = pl.pallas_call(
        _mid_body,
        out_shape=jax.ShapeDtypeStruct((N, D), jnp.float32),
    )(p1, g1, dinv, b1.reshape(1, D), W2)

    p2 = agg(g2, src, dst)

    out = pl.pallas_call(
        _final_body,
        out_shape=jax.ShapeDtypeStruct((G, D), jnp.float32),
    )(p2, g2, dinv, b2.reshape(1, D), batch.reshape(1, N), Wfc, bfc.reshape(1, D))
    return out
